# baseline (speedup 1.0000x reference)
import functools

import jax
import jax.numpy as jnp
from jax import lax
from jax.experimental import pallas as pl
from jax.experimental.pallas import tpu as pltpu
from jax.experimental.pallas import tpu_sc as plsc

_B, _T = 1024, 50
_BT = _B * _T
_NGRAMS = 16
_VAL_DIM = 64
_KEY_DIM = 32
_TOKEN_DIM = 128
_KPAD = 1024
_LPAD = 128

_NC, _NS = 2, 16
_NW = _NC * _NS
_TOK_W = _BT // _NW
_CHUNK = 16
_STEPS = _TOK_W // _CHUNK
_IPC = _CHUNK * _NGRAMS
_IDXV = 128
_NDMA = _IPC // _IDXV
_VPAD = 2 * _VAL_DIM


_PBLK = 8192
_PGRID = 1_000_000 // _PBLK


def _tc_pad_body(vt_ref, out_ref):
    out_ref[:, : _VAL_DIM] = vt_ref[...]


def _tc_pad(vt):
    return pl.pallas_call(
        _tc_pad_body,
        grid=(125,),
        in_specs=[pl.BlockSpec((8000, _VAL_DIM), lambda i: (i, 0))],
        out_specs=pl.BlockSpec((8000, _VPAD), lambda i: (i, 0)),
        out_shape=jax.ShapeDtypeStruct((1_000_000, _VPAD), jnp.float32),
    )(vt)


@functools.partial(
    pl.kernel,
    mesh=plsc.VectorSubcoreMesh(core_axis_name="c", subcore_axis_name="s"),
    out_type=jax.ShapeDtypeStruct((_BT, _VAL_DIM), jnp.float32),
    scratch_types=[
        pltpu.VMEM((2, _NDMA, _IDXV), jnp.int32),
        pltpu.VMEM((2, _IPC, _VPAD), jnp.float32),
        pltpu.VMEM((_CHUNK, _VAL_DIM), jnp.float32),
        pltpu.SemaphoreType.DMA,
        pltpu.SemaphoreType.DMA,
    ],
)
def _sc_bag(ngram_hbm, vtab_hbm, out_hbm, idx_v, rows_v, acc_v, sem0, sem1):
    wid = lax.axis_index("s") * _NC + lax.axis_index("c")
    sems = (sem0, sem1)

    def fire(g, b, sem):
        pltpu.sync_copy(ngram_hbm.at[g], idx_v.at[b])
        for j in range(_NDMA):
            pltpu.async_copy(
                vtab_hbm.at[idx_v.at[b, j]],
                rows_v.at[b, pl.ds(j * _IDXV, _IDXV)],
                sem,
            )

    def drain(b, sem):
        for j in range(_NDMA):
            pltpu.make_async_copy(
                vtab_hbm.at[idx_v.at[b, j]],
                rows_v.at[b, pl.ds(j * _IDXV, _IDXV)],
                sem,
            ).wait()

    def reduce_store(g, b):
        rv = rows_v.at[b]

        def tok(t, carry):
            base = t * _NGRAMS
            for c in range(_VAL_DIM // 16):
                sl = pl.ds(c * 16, 16)
                a = rv[base, sl]
                for r in range(1, _NGRAMS):
                    a = a + rv[base + r, sl]
                acc_v[t, sl] = a * (1.0 / _NGRAMS)
            return carry

        lax.fori_loop(0, _CHUNK, tok, 0)
        pltpu.sync_copy(acc_v, out_hbm.at[pl.ds(g * _CHUNK, _CHUNK)])

    g0 = wid * _STEPS
    fire(g0, 0, sem0)
    fire(g0 + 1, 1, sem1)

    def super_step(s2, carry):
        for b in range(2):
            g = g0 + 2 * s2 + b
            drain(b, sems[b])
            reduce_store(g, b)

            @pl.when(s2 < _STEPS // 2 - 1)
            def _():
                fire(g + 2, b, sems[b])

        return carry

    lax.fori_loop(0, _STEPS // 2, super_step, 0)


_BLK = 512
_GRID = _BT // _BLK


def _tc_pre_body(kidx_ref, lidx_ref, ktab_ref, ltab_ref, wk_ref, b_ref,
                 out_ref):
    kidx = kidx_ref[...]
    lidx = lidx_ref[...]
    onek = (lax.broadcasted_iota(jnp.int32, (_BLK, _KPAD), 1)
            == kidx).astype(jnp.float32)
    onel = (lax.broadcasted_iota(jnp.int32, (_BLK, _LPAD), 1)
            == lidx).astype(jnp.float32)
    kemb = jnp.dot(onek, ktab_ref[...],
                   preferred_element_type=jnp.float32)
    lemb = jnp.dot(onel, ltab_ref[...],
                   preferred_element_type=jnp.float32)
    out_ref[...] = (jnp.dot(kemb, wk_ref[...],
                            preferred_element_type=jnp.float32)
                    + b_ref[...] + lemb)


def _tc_pre(kidx, lidx, ktab, ltab, wk, b):
    return pl.pallas_call(
        _tc_pre_body,
        grid=(_GRID,),
        in_specs=[
            pl.BlockSpec((_BLK, 1), lambda i: (i, 0)),
            pl.BlockSpec((_BLK, 1), lambda i: (i, 0)),
            pl.BlockSpec((_KPAD, _KEY_DIM), lambda i: (0, 0)),
            pl.BlockSpec((_LPAD, _TOKEN_DIM), lambda i: (0, 0)),
            pl.BlockSpec((_KEY_DIM, _TOKEN_DIM), lambda i: (0, 0)),
            pl.BlockSpec((1, _TOKEN_DIM), lambda i: (0, 0)),
        ],
        out_specs=pl.BlockSpec((_BLK, _TOKEN_DIM), lambda i: (i, 0)),
        out_shape=jax.ShapeDtypeStruct((_BT, _TOKEN_DIM), jnp.float32),
    )(kidx, lidx, ktab, ltab, wk, b)


def _tc_post_body(part_ref, val_ref, wv_ref, out_ref):
    out_ref[...] = part_ref[...] + jnp.dot(
        val_ref[...], wv_ref[...], preferred_element_type=jnp.float32)


def _tc_post(part, val_emb, wv):
    return pl.pallas_call(
        _tc_post_body,
        grid=(_GRID,),
        in_specs=[
            pl.BlockSpec((_BLK, _TOKEN_DIM), lambda i: (i, 0)),
            pl.BlockSpec((_BLK, _VAL_DIM), lambda i: (i, 0)),
            pl.BlockSpec((_VAL_DIM, _TOKEN_DIM), lambda i: (0, 0)),
        ],
        out_specs=pl.BlockSpec((_BLK, _TOKEN_DIM), lambda i: (i, 0)),
        out_shape=jax.ShapeDtypeStruct((_BT, _TOKEN_DIM), jnp.float32),
    )(part, val_emb, wv)


def kernel(key_idx, landmark_idx, ngram_indices, key_table, value_table,
           landmark_table, W_proj, b_proj):
    ngram3 = ngram_indices.reshape(_NW * _STEPS, _NDMA, _IDXV)
    vpad = _tc_pad(value_table)
    val_emb = _sc_bag(ngram3, vpad)

    kidx = key_idx.reshape(_BT, 1)
    lidx = landmark_idx.reshape(_BT, 1)
    ktab = jnp.zeros((_KPAD, _KEY_DIM), jnp.float32).at[
        :key_table.shape[0]].set(key_table)
    ltab = jnp.zeros((_LPAD, _TOKEN_DIM), jnp.float32).at[
        :landmark_table.shape[0]].set(landmark_table)
    part = _tc_pre(kidx, lidx, ktab, ltab, W_proj[:_KEY_DIM],
                   b_proj.reshape(1, _TOKEN_DIM))
    out = _tc_post(part, val_emb, W_proj[_KEY_DIM:])
    return out.reshape(_B, _T, _TOKEN_DIM)


# speedup vs baseline: 1.1561x; 1.1561x over previous
import functools

import jax
import jax.numpy as jnp
from jax import lax
from jax.experimental import pallas as pl
from jax.experimental.pallas import tpu as pltpu
from jax.experimental.pallas import tpu_sc as plsc

_B, _T = 1024, 50
_BT = _B * _T
_NGRAMS = 16
_VAL_DIM = 64
_KEY_DIM = 32
_TOKEN_DIM = 128
_KPAD = 1024
_LPAD = 128

_NC, _NS = 2, 16
_NW = _NC * _NS
_TOK_W = _BT // _NW
_CHUNK = 16
_STEPS = _TOK_W // _CHUNK
_IPC = _CHUNK * _NGRAMS
_IDXV = 128
_NDMA = _IPC // _IDXV
_VPAD = 2 * _VAL_DIM


@functools.partial(
    pl.kernel,
    mesh=plsc.VectorSubcoreMesh(core_axis_name="c", subcore_axis_name="s"),
    out_type=jax.ShapeDtypeStruct((_BT, _VAL_DIM), jnp.float32),
    scratch_types=[
        pltpu.VMEM((2, _NDMA, _IDXV), jnp.int32),
        pltpu.VMEM((2, _IPC, _VPAD), jnp.float32),
        pltpu.VMEM((_CHUNK, _VAL_DIM), jnp.float32),
        pltpu.SemaphoreType.DMA,
        pltpu.SemaphoreType.DMA,
    ],
)
def _sc_bag(ngram_hbm, vtab_hbm, out_hbm, idx_v, rows_v, acc_v, sem0, sem1):
    wid = lax.axis_index("s") * _NC + lax.axis_index("c")
    sems = (sem0, sem1)

    def fire(g, b, sem):
        pltpu.sync_copy(ngram_hbm.at[g], idx_v.at[b])
        for j in range(_NDMA):
            pltpu.async_copy(
                vtab_hbm.at[idx_v.at[b, j]],
                rows_v.at[b, pl.ds(j * _IDXV, _IDXV)],
                sem,
            )

    def drain(b, sem):
        for j in range(_NDMA):
            pltpu.make_async_copy(
                vtab_hbm.at[idx_v.at[b, j]],
                rows_v.at[b, pl.ds(j * _IDXV, _IDXV)],
                sem,
            ).wait()

    def reduce_store(g, b):
        rv = rows_v.at[b]

        def tok(t, carry):
            base = t * _NGRAMS
            for c in range(_VAL_DIM // 16):
                sl = pl.ds(c * 16, 16)
                a = rv[base, sl]
                for r in range(1, _NGRAMS):
                    a = a + rv[base + r, sl]
                acc_v[t, sl] = a * (1.0 / _NGRAMS)
            return carry

        lax.fori_loop(0, _CHUNK, tok, 0)
        pltpu.sync_copy(acc_v, out_hbm.at[pl.ds(g * _CHUNK, _CHUNK)])

    g0 = wid * _STEPS
    fire(g0, 0, sem0)
    fire(g0 + 1, 1, sem1)

    def super_step(s2, carry):
        for b in range(2):
            g = g0 + 2 * s2 + b
            drain(b, sems[b])
            reduce_store(g, b)

            @pl.when(s2 < _STEPS // 2 - 1)
            def _():
                fire(g + 2, b, sems[b])

        return carry

    lax.fori_loop(0, _STEPS // 2, super_step, 0)


_BLK = 512
_GRID = _BT // _BLK


def _tc_pre_body(kidx_ref, lidx_ref, ktab_ref, ltab_ref, wk_ref, b_ref,
                 out_ref):
    kidx = kidx_ref[...]
    lidx = lidx_ref[...]
    onek = (lax.broadcasted_iota(jnp.int32, (_BLK, _KPAD), 1)
            == kidx).astype(jnp.float32)
    onel = (lax.broadcasted_iota(jnp.int32, (_BLK, _LPAD), 1)
            == lidx).astype(jnp.float32)
    kemb = jnp.dot(onek, ktab_ref[...],
                   preferred_element_type=jnp.float32)
    lemb = jnp.dot(onel, ltab_ref[...],
                   preferred_element_type=jnp.float32)
    out_ref[...] = (jnp.dot(kemb, wk_ref[...],
                            preferred_element_type=jnp.float32)
                    + b_ref[...] + lemb)


def _tc_pre(kidx, lidx, ktab, ltab, wk, b):
    return pl.pallas_call(
        _tc_pre_body,
        grid=(_GRID,),
        in_specs=[
            pl.BlockSpec((_BLK, 1), lambda i: (i, 0)),
            pl.BlockSpec((_BLK, 1), lambda i: (i, 0)),
            pl.BlockSpec((_KPAD, _KEY_DIM), lambda i: (0, 0)),
            pl.BlockSpec((_LPAD, _TOKEN_DIM), lambda i: (0, 0)),
            pl.BlockSpec((_KEY_DIM, _TOKEN_DIM), lambda i: (0, 0)),
            pl.BlockSpec((1, _TOKEN_DIM), lambda i: (0, 0)),
        ],
        out_specs=pl.BlockSpec((_BLK, _TOKEN_DIM), lambda i: (i, 0)),
        out_shape=jax.ShapeDtypeStruct((_BT, _TOKEN_DIM), jnp.float32),
    )(kidx, lidx, ktab, ltab, wk, b)


def _tc_post_body(part_ref, val_ref, wv_ref, out_ref):
    out_ref[...] = part_ref[...] + jnp.dot(
        val_ref[...], wv_ref[...], preferred_element_type=jnp.float32)


def _tc_post(part, val_emb, wv):
    return pl.pallas_call(
        _tc_post_body,
        grid=(_GRID,),
        in_specs=[
            pl.BlockSpec((_BLK, _TOKEN_DIM), lambda i: (i, 0)),
            pl.BlockSpec((_BLK, _VAL_DIM), lambda i: (i, 0)),
            pl.BlockSpec((_VAL_DIM, _TOKEN_DIM), lambda i: (0, 0)),
        ],
        out_specs=pl.BlockSpec((_BLK, _TOKEN_DIM), lambda i: (i, 0)),
        out_shape=jax.ShapeDtypeStruct((_BT, _TOKEN_DIM), jnp.float32),
    )(part, val_emb, wv)


def kernel(key_idx, landmark_idx, ngram_indices, key_table, value_table,
           landmark_table, W_proj, b_proj):
    ngram3 = ngram_indices.reshape(_NW * _STEPS, _NDMA, _IDXV)
    vpad = jnp.pad(value_table, ((0, 0), (0, _VAL_DIM)))
    val_emb = _sc_bag(ngram3, vpad)

    kidx = key_idx.reshape(_BT, 1)
    lidx = landmark_idx.reshape(_BT, 1)
    ktab = jnp.zeros((_KPAD, _KEY_DIM), jnp.float32).at[
        :key_table.shape[0]].set(key_table)
    ltab = jnp.zeros((_LPAD, _TOKEN_DIM), jnp.float32).at[
        :landmark_table.shape[0]].set(landmark_table)
    part = _tc_pre(kidx, lidx, ktab, ltab, W_proj[:_KEY_DIM],
                   b_proj.reshape(1, _TOKEN_DIM))
    out = _tc_post(part, val_emb, W_proj[_KEY_DIM:])
    return out.reshape(_B, _T, _TOKEN_DIM)


# speedup vs baseline: 1.1610x; 1.0042x over previous
import functools

import jax
import jax.numpy as jnp
from jax import lax
from jax.experimental import pallas as pl
from jax.experimental.pallas import tpu as pltpu
from jax.experimental.pallas import tpu_sc as plsc

_B, _T = 1024, 50
_BT = _B * _T
_NGRAMS = 16
_VAL_DIM = 64
_KEY_DIM = 32
_TOKEN_DIM = 128
_KPAD = 1024
_LPAD = 128

_NC, _NS = 2, 16
_NW = _NC * _NS
_TOK_W = _BT // _NW
_CHUNK = 16
_STEPS = _TOK_W // _CHUNK
_IPC = _CHUNK * _NGRAMS
_IDXV = 128
_NDMA = _IPC // _IDXV
_VROW = 2 * _VAL_DIM


@functools.partial(
    pl.kernel,
    mesh=plsc.VectorSubcoreMesh(core_axis_name="c", subcore_axis_name="s"),
    out_type=jax.ShapeDtypeStruct((_BT, _VAL_DIM), jnp.float32),
    scratch_types=[
        pltpu.VMEM((2, _NDMA, _IDXV), jnp.int32),
        pltpu.VMEM((2, _IPC, _VROW), jnp.float32),
        pltpu.VMEM((_CHUNK, _VAL_DIM), jnp.float32),
        pltpu.SemaphoreType.DMA,
        pltpu.SemaphoreType.DMA,
    ],
)
def _sc_bag(ngram_hbm, vtab_hbm, out_hbm, idx_v, rows_v, acc_v, sem0, sem1):
    wid = lax.axis_index("s") * _NC + lax.axis_index("c")
    sems = (sem0, sem1)

    def fire(g, b, sem):
        pltpu.sync_copy(ngram_hbm.at[g], idx_v.at[b])
        for j in range(_NDMA):
            pltpu.async_copy(
                vtab_hbm.at[idx_v.at[b, j]],
                rows_v.at[b, pl.ds(j * _IDXV, _IDXV)],
                sem,
            )

    def drain(b, sem):
        for j in range(_NDMA):
            pltpu.make_async_copy(
                vtab_hbm.at[idx_v.at[b, j]],
                rows_v.at[b, pl.ds(j * _IDXV, _IDXV)],
                sem,
            ).wait()

    def reduce_store(g, b):
        rv = rows_v.at[b]

        def tok(t, carry):
            base = t * _NGRAMS
            for c in range(_VAL_DIM // 16):
                sl = pl.ds(c * 16, 16)
                a = rv[base, sl]
                for r in range(1, _NGRAMS):
                    a = a + rv[base + r, sl]
                acc_v[t, sl] = a
            return carry

        lax.fori_loop(0, _CHUNK, tok, 0)
        pltpu.sync_copy(acc_v, out_hbm.at[pl.ds(g * _CHUNK, _CHUNK)])

    g0 = wid * _STEPS
    fire(g0, 0, sem0)
    fire(g0 + 1, 1, sem1)

    def super_step(s2, carry):
        for b in range(2):
            g = g0 + 2 * s2 + b
            drain(b, sems[b])
            reduce_store(g, b)

            @pl.when(s2 < _STEPS // 2 - 1)
            def _():
                fire(g + 2, b, sems[b])

        return carry

    lax.fori_loop(0, _STEPS // 2, super_step, 0)


_BLK = 512
_GRID = _BT // _BLK


def _tc_pre_body(kidx_ref, lidx_ref, ktab_ref, ltab_ref, wk_ref, b_ref,
                 out_ref):
    kidx = kidx_ref[...]
    lidx = lidx_ref[...]
    onek = (lax.broadcasted_iota(jnp.int32, (_BLK, _KPAD), 1)
            == kidx).astype(jnp.float32)
    onel = (lax.broadcasted_iota(jnp.int32, (_BLK, _LPAD), 1)
            == lidx).astype(jnp.float32)
    kemb = jnp.dot(onek, ktab_ref[...],
                   preferred_element_type=jnp.float32)
    lemb = jnp.dot(onel, ltab_ref[...],
                   preferred_element_type=jnp.float32)
    out_ref[...] = (jnp.dot(kemb, wk_ref[...],
                            preferred_element_type=jnp.float32)
                    + b_ref[...] + lemb)


def _tc_pre(kidx, lidx, ktab, ltab, wk, b):
    return pl.pallas_call(
        _tc_pre_body,
        grid=(_GRID,),
        in_specs=[
            pl.BlockSpec((_BLK, 1), lambda i: (i, 0)),
            pl.BlockSpec((_BLK, 1), lambda i: (i, 0)),
            pl.BlockSpec((_KPAD, _KEY_DIM), lambda i: (0, 0)),
            pl.BlockSpec((_LPAD, _TOKEN_DIM), lambda i: (0, 0)),
            pl.BlockSpec((_KEY_DIM, _TOKEN_DIM), lambda i: (0, 0)),
            pl.BlockSpec((1, _TOKEN_DIM), lambda i: (0, 0)),
        ],
        out_specs=pl.BlockSpec((_BLK, _TOKEN_DIM), lambda i: (i, 0)),
        out_shape=jax.ShapeDtypeStruct((_BT, _TOKEN_DIM), jnp.float32),
    )(kidx, lidx, ktab, ltab, wk, b)


def _tc_post_body(part_ref, val_ref, wv_ref, out_ref):
    out_ref[...] = part_ref[...] + jnp.dot(
        val_ref[...], wv_ref[...], preferred_element_type=jnp.float32)


def _tc_post(part, val_emb, wv):
    return pl.pallas_call(
        _tc_post_body,
        grid=(_GRID,),
        in_specs=[
            pl.BlockSpec((_BLK, _TOKEN_DIM), lambda i: (i, 0)),
            pl.BlockSpec((_BLK, _VAL_DIM), lambda i: (i, 0)),
            pl.BlockSpec((_VAL_DIM, _TOKEN_DIM), lambda i: (0, 0)),
        ],
        out_specs=pl.BlockSpec((_BLK, _TOKEN_DIM), lambda i: (i, 0)),
        out_shape=jax.ShapeDtypeStruct((_BT, _TOKEN_DIM), jnp.float32),
    )(part, val_emb, wv)


def kernel(key_idx, landmark_idx, ngram_indices, key_table, value_table,
           landmark_table, W_proj, b_proj):
    ngram3 = ngram_indices.reshape(_NW * _STEPS, _NDMA, _IDXV)
    vpad = jnp.pad(value_table, ((0, 0), (0, _VAL_DIM)))
    val_emb = _sc_bag(ngram3, vpad)

    kidx = key_idx.reshape(_BT, 1)
    lidx = landmark_idx.reshape(_BT, 1)
    ktab = jnp.zeros((_KPAD, _KEY_DIM), jnp.float32).at[
        :key_table.shape[0]].set(key_table)
    ltab = jnp.zeros((_LPAD, _TOKEN_DIM), jnp.float32).at[
        :landmark_table.shape[0]].set(landmark_table)
    part = _tc_pre(kidx, lidx, ktab, ltab, W_proj[:_KEY_DIM],
                   b_proj.reshape(1, _TOKEN_DIM))
    out = _tc_post(part, val_emb, W_proj[_KEY_DIM:] * (1.0 / _NGRAMS))
    return out.reshape(_B, _T, _TOKEN_DIM)


# speedup vs baseline: 1.1938x; 1.0283x over previous
import functools

import jax
import jax.numpy as jnp
from jax import lax
from jax.experimental import pallas as pl
from jax.experimental.pallas import tpu as pltpu
from jax.experimental.pallas import tpu_sc as plsc

_B, _T = 1024, 50
_BT = _B * _T
_NGRAMS = 16
_VAL_DIM = 64
_KEY_DIM = 32
_TOKEN_DIM = 128
_KPAD = 1024
_LPAD = 128

_NC, _NS = 2, 16
_NW = _NC * _NS
_TOK_W = _BT // _NW
_CHUNK = 16
_STEPS = _TOK_W // _CHUNK
_IPC = _CHUNK * _NGRAMS
_IDXV = 128
_NDMA = _IPC // _IDXV
_VROW = 2 * _VAL_DIM


@functools.partial(
    pl.kernel,
    mesh=plsc.VectorSubcoreMesh(core_axis_name="c", subcore_axis_name="s"),
    out_type=jax.ShapeDtypeStruct((_BT, _VAL_DIM), jnp.float32),
    scratch_types=[
        pltpu.VMEM((_STEPS, _NDMA, _IDXV), jnp.int32),
        pltpu.VMEM((2, _IPC, _VROW), jnp.float32),
        pltpu.VMEM((_CHUNK, _VAL_DIM), jnp.float32),
        pltpu.SemaphoreType.DMA,
        pltpu.SemaphoreType.DMA,
    ],
)
def _sc_bag(ngram_hbm, vtab_hbm, out_hbm, idx_v, rows_v, acc_v, sem0, sem1):
    wid = lax.axis_index("s") * _NC + lax.axis_index("c")
    sems = (sem0, sem1)
    g0 = wid * _STEPS
    pltpu.sync_copy(ngram_hbm.at[pl.ds(g0, _STEPS)], idx_v)

    def fire(s, b, sem):
        for j in range(_NDMA):
            pltpu.async_copy(
                vtab_hbm.at[idx_v.at[s, j]],
                rows_v.at[b, pl.ds(j * _IDXV, _IDXV)],
                sem,
            )

    def drain(s, b, sem):
        for j in range(_NDMA):
            pltpu.make_async_copy(
                vtab_hbm.at[idx_v.at[s, j]],
                rows_v.at[b, pl.ds(j * _IDXV, _IDXV)],
                sem,
            ).wait()

    def reduce_store(g, b):
        rv = rows_v.at[b]

        def tok(t, carry):
            base = t * _NGRAMS
            for c in range(_VAL_DIM // 16):
                sl = pl.ds(c * 16, 16)
                a = rv[base, sl]
                for r in range(1, _NGRAMS):
                    a = a + rv[base + r, sl]
                acc_v[t, sl] = a
            return carry

        lax.fori_loop(0, _CHUNK, tok, 0)
        pltpu.sync_copy(acc_v, out_hbm.at[pl.ds(g * _CHUNK, _CHUNK)])

    fire(0, 0, sem0)
    fire(1, 1, sem1)

    def super_step(s2, carry):
        for b in range(2):
            s = 2 * s2 + b
            drain(s, b, sems[b])
            reduce_store(g0 + s, b)

            @pl.when(s2 < _STEPS // 2 - 1)
            def _():
                fire(s + 2, b, sems[b])

        return carry

    lax.fori_loop(0, _STEPS // 2, super_step, 0)


_BLK = 512
_GRID = _BT // _BLK


def _tc_pre_body(kidx_ref, lidx_ref, ktab_ref, ltab_ref, wk_ref, b_ref,
                 out_ref):
    kidx = kidx_ref[...]
    lidx = lidx_ref[...]
    onek = (lax.broadcasted_iota(jnp.int32, (_BLK, _KPAD), 1)
            == kidx).astype(jnp.float32)
    onel = (lax.broadcasted_iota(jnp.int32, (_BLK, _LPAD), 1)
            == lidx).astype(jnp.float32)
    kemb = jnp.dot(onek, ktab_ref[...],
                   preferred_element_type=jnp.float32)
    lemb = jnp.dot(onel, ltab_ref[...],
                   preferred_element_type=jnp.float32)
    out_ref[...] = (jnp.dot(kemb, wk_ref[...],
                            preferred_element_type=jnp.float32)
                    + b_ref[...] + lemb)


def _tc_pre(kidx, lidx, ktab, ltab, wk, b):
    return pl.pallas_call(
        _tc_pre_body,
        grid=(_GRID,),
        in_specs=[
            pl.BlockSpec((_BLK, 1), lambda i: (i, 0)),
            pl.BlockSpec((_BLK, 1), lambda i: (i, 0)),
            pl.BlockSpec((_KPAD, _KEY_DIM), lambda i: (0, 0)),
            pl.BlockSpec((_LPAD, _TOKEN_DIM), lambda i: (0, 0)),
            pl.BlockSpec((_KEY_DIM, _TOKEN_DIM), lambda i: (0, 0)),
            pl.BlockSpec((1, _TOKEN_DIM), lambda i: (0, 0)),
        ],
        out_specs=pl.BlockSpec((_BLK, _TOKEN_DIM), lambda i: (i, 0)),
        out_shape=jax.ShapeDtypeStruct((_BT, _TOKEN_DIM), jnp.float32),
    )(kidx, lidx, ktab, ltab, wk, b)


def _tc_post_body(part_ref, val_ref, wv_ref, out_ref):
    out_ref[...] = part_ref[...] + jnp.dot(
        val_ref[...], wv_ref[...], preferred_element_type=jnp.float32)


def _tc_post(part, val_emb, wv):
    return pl.pallas_call(
        _tc_post_body,
        grid=(_GRID,),
        in_specs=[
            pl.BlockSpec((_BLK, _TOKEN_DIM), lambda i: (i, 0)),
            pl.BlockSpec((_BLK, _VAL_DIM), lambda i: (i, 0)),
            pl.BlockSpec((_VAL_DIM, _TOKEN_DIM), lambda i: (0, 0)),
        ],
        out_specs=pl.BlockSpec((_BLK, _TOKEN_DIM), lambda i: (i, 0)),
        out_shape=jax.ShapeDtypeStruct((_BT, _TOKEN_DIM), jnp.float32),
    )(part, val_emb, wv)


def kernel(key_idx, landmark_idx, ngram_indices, key_table, value_table,
           landmark_table, W_proj, b_proj):
    ngram3 = ngram_indices.reshape(_NW * _STEPS, _NDMA, _IDXV)
    vpad = jnp.pad(value_table, ((0, 0), (0, _VAL_DIM)))
    val_emb = _sc_bag(ngram3, vpad)

    kidx = key_idx.reshape(_BT, 1)
    lidx = landmark_idx.reshape(_BT, 1)
    ktab = jnp.zeros((_KPAD, _KEY_DIM), jnp.float32).at[
        :key_table.shape[0]].set(key_table)
    ltab = jnp.zeros((_LPAD, _TOKEN_DIM), jnp.float32).at[
        :landmark_table.shape[0]].set(landmark_table)
    part = _tc_pre(kidx, lidx, ktab, ltab, W_proj[:_KEY_DIM],
                   b_proj.reshape(1, _TOKEN_DIM))
    out = _tc_post(part, val_emb, W_proj[_KEY_DIM:] * (1.0 / _NGRAMS))
    return out.reshape(_B, _T, _TOKEN_DIM)
